# baseline (speedup 1.0000x reference)
import functools

import jax
import jax.numpy as jnp
from jax import lax
from jax.experimental import pallas as pl
from jax.experimental.pallas import tpu as pltpu
from jax.experimental.pallas import tpu_sc as plsc

_B, _C, _H, _W = 8, 192, 224, 224
_ROWS = _B * _C
_D = _H * _W
_NC, _NS = 2, 16
_NW = _NC * _NS
_RPW = _ROWS // _NW

_mesh = plsc.VectorSubcoreMesh(core_axis_name="c", subcore_axis_name="s")


@functools.partial(
    pl.kernel,
    mesh=_mesh,
    out_type=jax.ShapeDtypeStruct((_ROWS, _D), jnp.float32),
)
def _reverse_rows(in_hbm, out_hbm):
    cid = lax.axis_index("c")
    sid = lax.axis_index("s")
    wid = sid * _NC + cid
    base = wid * _RPW

    def body(i, carry):
        r = base + i
        b = r // _C
        c = r % _C
        src = b * _C + (_C - 1 - c)
        pltpu.sync_copy(in_hbm.at[src], out_hbm.at[r])
        return carry

    lax.fori_loop(0, _RPW, body, 0)


def kernel(input):
    x = input.reshape(_ROWS, _D)
    y = _reverse_rows(x)
    return y.reshape(_B, _C, _H, _W)


# speedup vs baseline: 11.3353x; 11.3353x over previous
import functools

import jax
import jax.numpy as jnp
from jax import lax
from jax.experimental import pallas as pl
from jax.experimental.pallas import tpu as pltpu
from jax.experimental.pallas import tpu_sc as plsc

_B, _C, _H, _W = 8, 192, 224, 224
_ROWS = _B * _C
_D = _H * _W
_NC, _NS = 2, 16
_NW = _NC * _NS
_RPW = _ROWS // _NW

_mesh = plsc.VectorSubcoreMesh(core_axis_name="c", subcore_axis_name="s")


@functools.partial(
    pl.kernel,
    mesh=_mesh,
    out_type=jax.ShapeDtypeStruct((_ROWS, _D), jnp.float32),
    scratch_types=[
        pltpu.VMEM((2, _D), jnp.float32),
        pltpu.SemaphoreType.DMA((2,)),
        pltpu.SemaphoreType.DMA((2,)),
    ],
)
def _reverse_rows(in_hbm, out_hbm, bufs, in_sems, out_sems):
    cid = lax.axis_index("c")
    sid = lax.axis_index("s")
    wid = sid * _NC + cid
    base = wid * _RPW

    def src_of(r):
        b = r // _C
        c = r % _C
        return b * _C + (_C - 1 - c)

    def start_in(i):
        slot = lax.rem(i, 2)
        pltpu.async_copy(in_hbm.at[src_of(base + i)], bufs.at[slot],
                         in_sems.at[slot])

    def wait_in(i):
        slot = lax.rem(i, 2)
        pltpu.make_async_copy(in_hbm.at[src_of(base + i)], bufs.at[slot],
                              in_sems.at[slot]).wait()

    def start_out(i):
        slot = lax.rem(i, 2)
        pltpu.async_copy(bufs.at[slot], out_hbm.at[base + i],
                         out_sems.at[slot])

    def wait_out(i):
        slot = lax.rem(i, 2)
        pltpu.make_async_copy(bufs.at[slot], out_hbm.at[base + i],
                              out_sems.at[slot]).wait()

    start_in(0)

    def body(i, carry):
        pl.when(i + 1 < _RPW)(lambda: start_in(i + 1))
        wait_in(i)
        pl.when(i >= 2)(lambda: wait_out(i - 2))
        start_out(i)
        return carry

    lax.fori_loop(0, _RPW, body, 0)
    wait_out(_RPW - 2)
    wait_out(_RPW - 1)


def kernel(input):
    x = input.reshape(_ROWS, _D)
    y = _reverse_rows(x)
    return y.reshape(_B, _C, _H, _W)


# speedup vs baseline: 11.3526x; 1.0015x over previous
import functools

import jax
import jax.numpy as jnp
from jax import lax
from jax.experimental import pallas as pl
from jax.experimental.pallas import tpu as pltpu
from jax.experimental.pallas import tpu_sc as plsc

_B, _C, _H, _W = 8, 192, 224, 224
_ROWS = _B * _C
_D = _H * _W
_NC, _NS = 2, 16
_NW = _NC * _NS
_RPW = _ROWS // _NW

_mesh = plsc.VectorSubcoreMesh(core_axis_name="c", subcore_axis_name="s")

_SPLIT = 2
_CH = _D // _SPLIT
_NBUF = 4
_T = _RPW * _SPLIT


@functools.partial(
    pl.kernel,
    mesh=_mesh,
    out_type=jax.ShapeDtypeStruct((_ROWS, _D), jnp.float32),
    scratch_types=[
        pltpu.VMEM((_NBUF, _CH), jnp.float32),
        pltpu.SemaphoreType.DMA((_NBUF,)),
        pltpu.SemaphoreType.DMA((_NBUF,)),
    ],
)
def _reverse_rows(in_hbm, out_hbm, bufs, in_sems, out_sems):
    cid = lax.axis_index("c")
    sid = lax.axis_index("s")
    wid = sid * _NC + cid
    base = wid * _RPW

    def src_slice(i):
        r = base + i // _SPLIT
        k = lax.rem(i, _SPLIT)
        b = r // _C
        c = lax.rem(r, _C)
        src = b * _C + (_C - 1 - c)
        return in_hbm.at[src, pl.ds(k * _CH, _CH)]

    def dst_slice(i):
        r = base + i // _SPLIT
        k = lax.rem(i, _SPLIT)
        return out_hbm.at[r, pl.ds(k * _CH, _CH)]

    def start_in(i):
        slot = lax.rem(i, _NBUF)
        pltpu.async_copy(src_slice(i), bufs.at[slot], in_sems.at[slot])

    def wait_in(i):
        slot = lax.rem(i, _NBUF)
        pltpu.make_async_copy(src_slice(i), bufs.at[slot],
                              in_sems.at[slot]).wait()

    def start_out(i):
        slot = lax.rem(i, _NBUF)
        pltpu.async_copy(bufs.at[slot], dst_slice(i), out_sems.at[slot])

    def wait_out(i):
        slot = lax.rem(i, _NBUF)
        pltpu.make_async_copy(bufs.at[slot], dst_slice(i),
                              out_sems.at[slot]).wait()

    for j in range(_NBUF - 1):
        start_in(j)

    def body(i, carry):
        wait_in(i)
        start_out(i)
        pl.when(jnp.logical_and(i >= 1, i + _NBUF - 1 < _T))(
            lambda: wait_out(i - 1))
        pl.when(i + _NBUF - 1 < _T)(lambda: start_in(i + _NBUF - 1))
        return carry

    lax.fori_loop(0, _T, body, 0)
    for j in range(_T - _NBUF, _T):
        wait_out(j)


def kernel(input):
    x = input.reshape(_ROWS, _D)
    y = _reverse_rows(x)
    return y.reshape(_B, _C, _H, _W)


# speedup vs baseline: 11.6104x; 1.0227x over previous
import functools

import jax
import jax.numpy as jnp
from jax import lax
from jax.experimental import pallas as pl
from jax.experimental.pallas import tpu as pltpu
from jax.experimental.pallas import tpu_sc as plsc

_B, _C, _H, _W = 8, 192, 224, 224
_ROWS = _B * _C
_D = _H * _W
_NC, _NS = 2, 16
_NW = _NC * _NS
_RPW = _ROWS // _NW

_mesh = plsc.VectorSubcoreMesh(core_axis_name="c", subcore_axis_name="s")

_SPLIT = 2
_CH = _D // _SPLIT
_NBUF = 4
_T = _RPW * _SPLIT


@functools.partial(
    pl.kernel,
    mesh=_mesh,
    out_type=jax.ShapeDtypeStruct((_ROWS, _D), jnp.float32),
    scratch_types=[
        pltpu.VMEM_SHARED((_NS, _NBUF, _CH), jnp.float32),
        pltpu.SemaphoreType.DMA((_NBUF,)),
        pltpu.SemaphoreType.DMA((_NBUF,)),
    ],
)
def _reverse_rows(in_hbm, out_hbm, shared, in_sems, out_sems):
    cid = lax.axis_index("c")
    sid = lax.axis_index("s")
    wid = sid * _NC + cid
    base = wid * _RPW
    bufs = shared.at[sid]

    def src_slice(i):
        r = base + i // _SPLIT
        k = lax.rem(i, _SPLIT)
        b = r // _C
        c = lax.rem(r, _C)
        src = b * _C + (_C - 1 - c)
        return in_hbm.at[src, pl.ds(k * _CH, _CH)]

    def dst_slice(i):
        r = base + i // _SPLIT
        k = lax.rem(i, _SPLIT)
        return out_hbm.at[r, pl.ds(k * _CH, _CH)]

    def start_in(i):
        slot = lax.rem(i, _NBUF)
        pltpu.async_copy(src_slice(i), bufs.at[slot], in_sems.at[slot])

    def wait_in(i):
        slot = lax.rem(i, _NBUF)
        pltpu.make_async_copy(src_slice(i), bufs.at[slot],
                              in_sems.at[slot]).wait()

    def start_out(i):
        slot = lax.rem(i, _NBUF)
        pltpu.async_copy(bufs.at[slot], dst_slice(i), out_sems.at[slot])

    def wait_out(i):
        slot = lax.rem(i, _NBUF)
        pltpu.make_async_copy(bufs.at[slot], dst_slice(i),
                              out_sems.at[slot]).wait()

    for j in range(_NBUF - 1):
        start_in(j)

    def body(i, carry):
        wait_in(i)
        start_out(i)
        pl.when(jnp.logical_and(i >= 1, i + _NBUF - 1 < _T))(
            lambda: wait_out(i - 1))
        pl.when(i + _NBUF - 1 < _T)(lambda: start_in(i + _NBUF - 1))
        return carry

    lax.fori_loop(0, _T, body, 0)
    for j in range(_T - _NBUF, _T):
        wait_out(j)


def kernel(input):
    x = input.reshape(_ROWS, _D)
    y = _reverse_rows(x)
    return y.reshape(_B, _C, _H, _W)
